# baseline (speedup 1.0000x reference)
import functools
import jax, jax.numpy as jnp
from jax import lax
from jax.experimental import pallas as pl
from jax.experimental.pallas import tpu as pltpu
from jax.experimental.pallas import tpu_sc as plsc

R, C, B, NC, NS, BLK = 1024, 100000, 128, 2, 16, 8
PER_SC = (R // BLK) // NC

def _body(x_hbm, idx_hbm, out_hbm, buf0, idx_v, out_v):
    c = lax.axis_index("c")
    s = lax.axis_index("s")

    def do_chunk(k):
        @pl.when(s == 0)
        def _fill():
            row0 = (k * NC + c) * BLK
            pltpu.sync_copy(x_hbm.at[pl.ds(row0, BLK)], buf0)
        plsc.subcore_barrier()

    pl.loop(0, PER_SC)(do_chunk)
    wid = s * NC + c
    el0 = wid * (R * B // (NC * NS))
    pltpu.sync_copy(idx_hbm.at[pl.ds(el0, R * B // 32)], idx_v)
    for g in range(R * B // 32 // 16):
        sl = pl.ds(g * 16, 16)
        out_v[sl] = idx_v[sl].astype(jnp.float32)
    pltpu.sync_copy(out_v, out_hbm.at[pl.ds(el0, R * B // 32)])

def kernel(x, index):
    idx_flat = index.reshape(-1)
    mesh = plsc.VectorSubcoreMesh(core_axis_name="c", subcore_axis_name="s")
    run = functools.partial(
        pl.kernel, mesh=mesh,
        compiler_params=pltpu.CompilerParams(needs_layout_passes=False),
        out_type=jax.ShapeDtypeStruct((R * B,), jnp.float32),
        scratch_types=[
            pltpu.VMEM_SHARED((BLK, C), jnp.float32),
            pltpu.VMEM((R * B // 32,), jnp.int32),
            pltpu.VMEM((R * B // 32,), jnp.float32),
        ],
    )(_body)
    return run(x, idx_flat).reshape(R, B)


# speedup vs baseline: 1.0043x; 1.0043x over previous
import functools
import jax, jax.numpy as jnp
from jax import lax
from jax.experimental import pallas as pl
from jax.experimental.pallas import tpu as pltpu
from jax.experimental.pallas import tpu_sc as plsc

R, C, B, NC, NS, BLK = 1024, 100000, 128, 2, 16, 8
PER_SC = (R // BLK) // NC

def _body(x_hbm, idx_hbm, out_hbm, buf0, idx_v, out_v):
    c = lax.axis_index("c")
    s = lax.axis_index("s")

    def do_chunk(k):
        @pl.when(s == 0)
        def _fill():
            row0 = (k * NC + c) * BLK
            pltpu.sync_copy(x_hbm.at[pl.ds(row0, BLK)], buf0)

    pl.loop(0, PER_SC)(do_chunk)
    wid = s * NC + c
    el0 = wid * (R * B // (NC * NS))
    pltpu.sync_copy(idx_hbm.at[pl.ds(el0, R * B // 32)], idx_v)
    for g in range(R * B // 32 // 16):
        sl = pl.ds(g * 16, 16)
        out_v[sl] = idx_v[sl].astype(jnp.float32)
    pltpu.sync_copy(out_v, out_hbm.at[pl.ds(el0, R * B // 32)])

def kernel(x, index):
    idx_flat = index.reshape(-1)
    mesh = plsc.VectorSubcoreMesh(core_axis_name="c", subcore_axis_name="s")
    run = functools.partial(
        pl.kernel, mesh=mesh,
        compiler_params=pltpu.CompilerParams(needs_layout_passes=False),
        out_type=jax.ShapeDtypeStruct((R * B,), jnp.float32),
        scratch_types=[
            pltpu.VMEM_SHARED((BLK, C), jnp.float32),
            pltpu.VMEM((R * B // 32,), jnp.int32),
            pltpu.VMEM((R * B // 32,), jnp.float32),
        ],
    )(_body)
    return run(x, idx_flat).reshape(R, B)


# speedup vs baseline: 1.0235x; 1.0191x over previous
import functools
import jax, jax.numpy as jnp
from jax import lax
from jax.experimental import pallas as pl
from jax.experimental.pallas import tpu as pltpu
from jax.experimental.pallas import tpu_sc as plsc

R, C, B = 1024, 100000, 128
CP = 100352
NREP = 32

def _body(x_hbm, idx_hbm, out_hbm, buf_v, idx_v, out_v):
    c = lax.axis_index("c")
    s = lax.axis_index("s")
    wid = s * 2 + c

    def do_rep(k):
        src0 = (k * 7919 + wid * 31) % (R * B - CP)
        src0 = (src0 // 8) * 8
        pltpu.sync_copy(idx_hbm.at[pl.ds(src0, CP)], buf_v)

    pl.loop(0, NREP)(do_rep)
    el0 = wid * (R * B // 32)
    pltpu.sync_copy(idx_hbm.at[pl.ds(el0, R * B // 32)], idx_v)
    for g in range(R * B // 32 // 16):
        sl = pl.ds(g * 16, 16)
        out_v[sl] = idx_v[sl].astype(jnp.float32)
    pltpu.sync_copy(out_v, out_hbm.at[pl.ds(el0, R * B // 32)])

def kernel(x, index):
    idx_flat = index.reshape(-1)
    mesh = plsc.VectorSubcoreMesh(core_axis_name="c", subcore_axis_name="s")
    run = functools.partial(
        pl.kernel, mesh=mesh,
        compiler_params=pltpu.CompilerParams(needs_layout_passes=False),
        out_type=jax.ShapeDtypeStruct((R * B,), jnp.float32),
        scratch_types=[
            pltpu.VMEM((CP,), jnp.int32),
            pltpu.VMEM((R * B // 32,), jnp.int32),
            pltpu.VMEM((R * B // 32,), jnp.float32),
        ],
    )(_body)
    return run(x, idx_flat).reshape(R, B)


# speedup vs baseline: 1.2150x; 1.1871x over previous
import functools

import jax
import jax.numpy as jnp
from jax import lax
from jax.experimental import pallas as pl
from jax.experimental.pallas import tpu as pltpu
from jax.experimental.pallas import tpu_sc as plsc

R = 1024
C = 100000
B = 128
L = 16
NC = 2
NS = 16
NW = NC * NS
BLK = 8
NBLK = R // (BLK * NW)
GB = BLK * B
NWIN = 8
W = 12544
CMAIN = (C // 128) * 128
TAIL = C - CMAIN
LAST_START = CMAIN - W


def _body(x_hbm, tail_hbm, idx_hbm, out_hbm, idx_v, out_v, buf_v, tail_v):
    wid = lax.axis_index("s") * NC + lax.axis_index("c")

    def do_block(b):
        blk = wid * NBLK + b
        row0 = blk * BLK
        el0 = row0 * B
        pltpu.sync_copy(idx_hbm.at[pl.ds(el0, GB)], idx_v)
        pltpu.sync_copy(tail_hbm.at[pl.ds(row0, BLK)], tail_v)

        def do_window(k):
            start = pl.multiple_of(
                jnp.where(k == NWIN - 1, LAST_START, k * W), 128
            )
            lo = k * W
            hi = jnp.where(k == NWIN - 1, CMAIN, lo + W)
            pltpu.sync_copy(
                x_hbm.at[pl.ds(row0, BLK), pl.ds(start, W)], buf_v
            )
            for g in range(GB // L):
                sl = pl.ds(g * L, L)
                j = idx_v[sl]
                m = (j >= lo) & (j < hi)
                c = jnp.where(m, j - start, 0)
                rv = jnp.full((L,), g // (B // L), jnp.int32)
                got = plsc.load_gather(buf_v, [rv, c])
                out_v[sl] = jnp.where(m, got, out_v[sl])

        pl.loop(0, NWIN)(do_window)
        for g in range(GB // L):
            sl = pl.ds(g * L, L)
            j = idx_v[sl]
            m = j >= CMAIN
            c = jnp.where(m, j - CMAIN, 0)
            rv = jnp.full((L,), g // (B // L), jnp.int32)
            got = plsc.load_gather(tail_v, [rv, c])
            out_v[sl] = jnp.where(m, got, out_v[sl])
        pltpu.sync_copy(out_v, out_hbm.at[pl.ds(el0, GB)])

    pl.loop(0, NBLK)(do_block)


def kernel(x, index):
    x_tail = x[:, CMAIN:]
    idx_flat = index.reshape(-1)
    mesh = plsc.VectorSubcoreMesh(core_axis_name="c", subcore_axis_name="s")
    run = functools.partial(
        pl.kernel,
        mesh=mesh,
        compiler_params=pltpu.CompilerParams(needs_layout_passes=False),
        out_type=jax.ShapeDtypeStruct((R * B,), jnp.float32),
        scratch_types=[
            pltpu.VMEM((GB,), jnp.int32),
            pltpu.VMEM((GB,), jnp.float32),
            pltpu.VMEM((BLK, W), jnp.float32),
            pltpu.VMEM((BLK, TAIL), jnp.float32),
        ],
    )(_body)
    return run(x, x_tail, idx_flat).reshape(R, B)
